# baseline (speedup 1.0000x reference)
import functools

import jax
import jax.numpy as jnp
from jax.experimental import pallas as pl
from jax.experimental.pallas import tpu as pltpu

EPS_ = 1e-6
_DEPTH = 2


def _loss_kernel(p_hbm, g_hbm, w_ref, out_ref,
                 bpf_ref, bpb_ref, bgf_ref, bgb_ref,
                 sem_pf, sem_pb, sem_gf, sem_gb, *, B, C):
    HALF = B // 2

    def start_chunk(k, slot):
        pltpu.make_async_copy(
            p_hbm.at[pl.ds(k * C, C)], bpf_ref.at[slot], sem_pf.at[slot]
        ).start(priority=0)
        pltpu.make_async_copy(
            p_hbm.at[pl.ds((HALF + k) * C, C)], bpb_ref.at[slot],
            sem_pb.at[slot]
        ).start(priority=1)
        pltpu.make_async_copy(
            g_hbm.at[pl.ds(k * C, C)], bgf_ref.at[slot], sem_gf.at[slot]
        ).start(priority=1)
        pltpu.make_async_copy(
            g_hbm.at[pl.ds((HALF + k) * C, C)], bgb_ref.at[slot],
            sem_gb.at[slot]
        ).start(priority=0)

    def wait_chunk(slot):
        for buf, sem in ((bpf_ref, sem_pf), (bpb_ref, sem_pb),
                         (bgf_ref, sem_gf), (bgb_ref, sem_gb)):
            pltpu.make_async_copy(
                p_hbm.at[pl.ds(0, C)], buf.at[slot], sem.at[slot]
            ).wait()

    for c in range(_DEPTH - 1):
        start_chunk(c, c)

    def batch_stats(bp, bg, slot, w_vec):
        pos_rows = []
        s_rows = []
        mx_rows = []
        for r in range(C):
            p = bp[slot, r]
            g = bg[slot, r]
            pos_rows.append(jnp.sum(g * p, axis=0, keepdims=True))
            s_rows.append(jnp.sum(p, axis=0, keepdims=True))
            mx_rows.append(jnp.max(g, axis=0, keepdims=True))
        pos_m = jnp.concatenate(pos_rows, axis=0)
        s_m = jnp.concatenate(s_rows, axis=0)
        mx_m = jnp.concatenate(mx_rows, axis=0)
        pos_c = jnp.sum(pos_m, axis=1, keepdims=True)
        s_c = jnp.sum(s_m, axis=1, keepdims=True)
        mx_c = jnp.max(mx_m, axis=1, keepdims=True)
        ratio = (s_c - pos_c) / (pos_c + EPS_)
        contrib = jnp.where(mx_c != 0.0, ratio * w_vec, 0.0)
        vb = jnp.max(mx_c, axis=0, keepdims=True)
        return contrib, jnp.where(vb != 0.0, 1.0, 0.0)

    def body(step, carry):
        acc_t, acc_n = carry
        slot = jax.lax.rem(step, _DEPTH)

        @pl.when(step + _DEPTH - 1 < HALF)
        def _():
            start_chunk(step + _DEPTH - 1,
                        jax.lax.rem(step + _DEPTH - 1, _DEPTH))

        wait_chunk(slot)
        c1, v1 = batch_stats(bpf_ref, bgf_ref, slot, w_ref[step])
        c2, v2 = batch_stats(bpb_ref, bgb_ref, slot, w_ref[HALF + step])
        return acc_t + c1 + c2, acc_n + v1 + v2

    acc_t = jnp.zeros((C, 1), jnp.float32)
    acc_n = jnp.zeros((1, 1), jnp.float32)
    acc_t, acc_n = jax.lax.fori_loop(0, B // 2, body, (acc_t, acc_n))

    total = jnp.sum(acc_t, axis=0, keepdims=True)
    n = jnp.maximum(acc_n, 1.0)
    out_ref[...] = jnp.where(total == 0.0, 0.0, jnp.log(total) / n)


@jax.jit
def kernel(Y_pred, Y_gt, label):
    B, C, H, W = Y_pred.shape
    label32 = label.astype(jnp.int32)
    n_rows = B * C
    rows_hw = H * W // 128
    Yp = Y_pred.reshape(n_rows, rows_hw, 128)
    Yg = Y_gt.reshape(n_rows, rows_hw, 128)

    cls = jnp.arange(C, dtype=jnp.int32)
    w = jnp.where(label32[:, None] == cls[None, :],
                  jnp.float32(1.0), jnp.float32(1.0 / C))
    w3 = w.reshape(B, C, 1)

    out = pl.pallas_call(
        functools.partial(_loss_kernel, B=B, C=C),
        in_specs=[
            pl.BlockSpec(memory_space=pl.ANY),
            pl.BlockSpec(memory_space=pl.ANY),
            pl.BlockSpec(memory_space=pltpu.VMEM),
        ],
        out_specs=pl.BlockSpec(memory_space=pltpu.VMEM),
        out_shape=jax.ShapeDtypeStruct((1, 1), jnp.float32),
        scratch_shapes=[
            pltpu.VMEM((_DEPTH, C, rows_hw, 128), jnp.float32),
            pltpu.VMEM((_DEPTH, C, rows_hw, 128), jnp.float32),
            pltpu.VMEM((_DEPTH, C, rows_hw, 128), jnp.float32),
            pltpu.VMEM((_DEPTH, C, rows_hw, 128), jnp.float32),
            pltpu.SemaphoreType.DMA((_DEPTH,)),
            pltpu.SemaphoreType.DMA((_DEPTH,)),
            pltpu.SemaphoreType.DMA((_DEPTH,)),
            pltpu.SemaphoreType.DMA((_DEPTH,)),
        ],
        compiler_params=pltpu.CompilerParams(
            vmem_limit_bytes=40 * 1024 * 1024,
        ),
    )(Yp, Yg, w3)
    return out[0, 0]


# speedup vs baseline: 1.0158x; 1.0158x over previous
import functools

import jax
import jax.numpy as jnp
from jax.experimental import pallas as pl
from jax.experimental.pallas import tpu as pltpu

EPS_ = 1e-6
_DEPTH = 3


def _loss_kernel(p_hbm, g_hbm, label_ref, out_ref,
                 bp_ref, bg_ref, sem_p, sem_g, *, B, C):
    def start_chunk(chunk, slot):
        src_p = p_hbm.at[pl.ds(chunk * C, C)]
        src_g = g_hbm.at[pl.ds(chunk * C, C)]
        pltpu.make_async_copy(src_p, bp_ref.at[slot], sem_p.at[slot]).start(
            priority=0)
        pltpu.make_async_copy(src_g, bg_ref.at[slot], sem_g.at[slot]).start(
            priority=1)

    def wait_chunk(slot):
        pltpu.make_async_copy(
            p_hbm.at[pl.ds(0, C)], bp_ref.at[slot], sem_p.at[slot]
        ).wait()
        pltpu.make_async_copy(
            g_hbm.at[pl.ds(0, C)], bg_ref.at[slot], sem_g.at[slot]
        ).wait()

    for c in range(_DEPTH - 1):
        start_chunk(c, c)

    def body(step, carry):
        acc_t, acc_n = carry
        slot = jax.lax.rem(step, _DEPTH)

        @pl.when(step + _DEPTH - 1 < B)
        def _():
            start_chunk(step + _DEPTH - 1,
                        jax.lax.rem(step + _DEPTH - 1, _DEPTH))

        wait_chunk(slot)
        pos_rows = []
        s_rows = []
        mx_rows = []
        for r in range(C):
            p = bp_ref[slot, r]
            g = bg_ref[slot, r]
            pos_rows.append(jnp.sum(g * p, axis=0, keepdims=True))
            s_rows.append(jnp.sum(p, axis=0, keepdims=True))
            mx_rows.append(jnp.max(g, axis=0, keepdims=True))
        pos_m = jnp.concatenate(pos_rows, axis=0)
        s_m = jnp.concatenate(s_rows, axis=0)
        mx_m = jnp.concatenate(mx_rows, axis=0)
        pos_c = jnp.sum(pos_m, axis=1, keepdims=True)
        s_c = jnp.sum(s_m, axis=1, keepdims=True)
        mx_c = jnp.max(mx_m, axis=1, keepdims=True)
        ratio = (s_c - pos_c) / (pos_c + EPS_)
        cls = jax.lax.broadcasted_iota(jnp.int32, (C, 1), 0)
        w_vec = jnp.where(cls == label_ref[step], 1.0, 1.0 / C)
        contrib = jnp.where(mx_c != 0.0, ratio * w_vec, 0.0)
        vb = jnp.max(mx_c, axis=0, keepdims=True)
        acc_t = acc_t + contrib
        acc_n = acc_n + jnp.where(vb != 0.0, 1.0, 0.0)
        return acc_t, acc_n

    acc_t = jnp.zeros((C, 1), jnp.float32)
    acc_n = jnp.zeros((1, 1), jnp.float32)
    acc_t, acc_n = jax.lax.fori_loop(0, B, body, (acc_t, acc_n))

    total = jnp.sum(acc_t, axis=0, keepdims=True)
    n = jnp.maximum(acc_n, 1.0)
    out_ref[...] = jnp.where(total == 0.0, 0.0, jnp.log(total) / n)


@jax.jit
def kernel(Y_pred, Y_gt, label):
    B, C, H, W = Y_pred.shape
    label32 = label.astype(jnp.int32)
    n_rows = B * C
    rows_hw = H * W // 128
    Yp = Y_pred.reshape(n_rows, rows_hw, 128)
    Yg = Y_gt.reshape(n_rows, rows_hw, 128)

    out = pl.pallas_call(
        functools.partial(_loss_kernel, B=B, C=C),
        in_specs=[
            pl.BlockSpec(memory_space=pl.ANY),
            pl.BlockSpec(memory_space=pl.ANY),
            pl.BlockSpec(memory_space=pltpu.SMEM),
        ],
        out_specs=pl.BlockSpec(memory_space=pltpu.VMEM),
        out_shape=jax.ShapeDtypeStruct((1, 1), jnp.float32),
        scratch_shapes=[
            pltpu.VMEM((_DEPTH, C, rows_hw, 128), jnp.float32),
            pltpu.VMEM((_DEPTH, C, rows_hw, 128), jnp.float32),
            pltpu.SemaphoreType.DMA((_DEPTH,)),
            pltpu.SemaphoreType.DMA((_DEPTH,)),
        ],
        compiler_params=pltpu.CompilerParams(
            vmem_limit_bytes=40 * 1024 * 1024,
        ),
    )(Yp, Yg, label32)
    return out[0, 0]
